# baseline (speedup 1.0000x reference)
import functools

import jax
import jax.numpy as jnp
from jax import lax
from jax.experimental import pallas as pl
from jax.experimental.pallas import tpu as pltpu
from jax.experimental.pallas import tpu_sc as plsc

T = 16384
E = 64
D = 64
NC, NS = 2, 16
NW = NC * NS
TW = T // NW


def _unpermute_body(x_hbm, out_hbm, sem):
    wid = lax.axis_index("s") * NC + lax.axis_index("c")
    t0 = wid * TW

    def copy(j):
        return pltpu.make_async_copy(
            x_hbm.at[pl.ds(t0, TW), pl.ds(E - 1 - j, 1)],
            out_hbm.at[pl.ds(t0, TW), pl.ds(j, 1)],
            sem)

    for j in range(E):
        copy(j).start()
    for j in range(E):
        copy(j).wait()


def kernel(x):
    mesh = plsc.VectorSubcoreMesh(core_axis_name="c", subcore_axis_name="s")
    run = functools.partial(
        pl.kernel,
        mesh=mesh,
        out_type=jax.ShapeDtypeStruct((T, E, D), jnp.float32),
        scratch_types=[pltpu.SemaphoreType.DMA],
        compiler_params=pltpu.CompilerParams(use_tc_tiling_on_sc=False),
    )(_unpermute_body)
    return run(x)


# speedup vs baseline: 6.9910x; 6.9910x over previous
import functools

import jax
import jax.numpy as jnp
from jax.experimental import pallas as pl
from jax.experimental.pallas import tpu as pltpu

T = 16384
E = 64
D = 64
BT = 256


def _rev_body(x_ref, o_ref):
    ridx = 7 - jax.lax.broadcasted_iota(jnp.int32, (BT, 8, D), 1)
    o_ref[...] = jnp.take_along_axis(x_ref[...], ridx, axis=1)


def kernel(x):
    return pl.pallas_call(
        _rev_body,
        grid=(T // BT, E // 8),
        in_specs=[pl.BlockSpec((BT, 8, D), lambda i, j: (i, (E // 8 - 1) - j, 0))],
        out_specs=pl.BlockSpec((BT, 8, D), lambda i, j: (i, j, 0)),
        out_shape=jax.ShapeDtypeStruct((T, E, D), jnp.float32),
    )(x)


# speedup vs baseline: 14.6364x; 2.0936x over previous
import jax
import jax.numpy as jnp
from jax.experimental import pallas as pl
from jax.experimental.pallas import tpu as pltpu

T = 16384
E = 64
D = 64
WR = 32
W = 128
BT = 256


def _rev_body(x_ref, o_ref):
    ridx = 7 - jax.lax.broadcasted_iota(jnp.int32, (BT, 8, W), 1)
    for k in range(WR // 8):
        seg = x_ref[:, 8 * k:8 * (k + 1), :]
        seg = jnp.take_along_axis(seg, ridx, axis=1)
        seg = pltpu.roll(seg, W // 2, 2)
        o_ref[:, WR - 8 * (k + 1):WR - 8 * k, :] = seg


def kernel(x):
    x4 = x.reshape(T, WR, W)
    y4 = pl.pallas_call(
        _rev_body,
        grid=(T // BT,),
        in_specs=[pl.BlockSpec((BT, WR, W), lambda i: (i, 0, 0))],
        out_specs=pl.BlockSpec((BT, WR, W), lambda i: (i, 0, 0)),
        out_shape=jax.ShapeDtypeStruct((T, WR, W), jnp.float32),
    )(x4)
    return y4.reshape(T, E, D)


# speedup vs baseline: 14.7182x; 1.0056x over previous
import jax
import jax.numpy as jnp
from jax.experimental import pallas as pl
from jax.experimental.pallas import tpu as pltpu

T = 16384
E = 64
D = 64
WR = 32
W = 128
BT = 256
N = T // BT


def _flip_block(vbuf, obuf, b):
    ridx = 7 - jax.lax.broadcasted_iota(jnp.int32, (BT, 8, W), 1)
    for k in range(WR // 8):
        seg = vbuf[b, :, 8 * k:8 * (k + 1), :]
        seg = jnp.take_along_axis(seg, ridx, axis=1)
        seg = pltpu.roll(seg, W // 2, 2)
        obuf[b, :, WR - 8 * (k + 1):WR - 8 * k, :] = seg


def _body(x_hbm, o_hbm, vbuf, obuf, gsem, wsem):
    def copy_in(i, b):
        return pltpu.make_async_copy(
            x_hbm.at[pl.ds(i * BT, BT)], vbuf.at[b], gsem.at[b])

    def copy_out(i, b):
        return pltpu.make_async_copy(
            obuf.at[b], o_hbm.at[pl.ds(i * BT, BT)], wsem.at[b])

    def step(i, b, first=False, last=False):
        copy_in(i, b).wait()
        if not first:
            copy_out(i - 2, b).wait()
        _flip_block(vbuf, obuf, b)
        copy_out(i, b).start()
        if not last:
            copy_in(i + 2, b).start()

    copy_in(0, 0).start()
    copy_in(1, 1).start()
    step(0, 0, first=True)
    step(1, 1, first=True)

    def group(g, carry):
        i = 2 * g
        step(i, 0)
        step(i + 1, 1)
        return carry

    jax.lax.fori_loop(1, N // 2 - 1, group, 0)

    step(N - 2, 0, last=True)
    step(N - 1, 1, last=True)
    copy_out(N - 2, 0).wait()
    copy_out(N - 1, 1).wait()


def kernel(x):
    x4 = x.reshape(T, WR, W)
    y4 = pl.pallas_call(
        _body,
        in_specs=[pl.BlockSpec(memory_space=pltpu.HBM)],
        out_specs=pl.BlockSpec(memory_space=pltpu.HBM),
        out_shape=jax.ShapeDtypeStruct((T, WR, W), jnp.float32),
        scratch_shapes=[
            pltpu.VMEM((2, BT, WR, W), jnp.float32),
            pltpu.VMEM((2, BT, WR, W), jnp.float32),
            pltpu.SemaphoreType.DMA((2,)),
            pltpu.SemaphoreType.DMA((2,)),
        ],
    )(x4)
    return y4.reshape(T, E, D)
